# Initial kernel scaffold; baseline (speedup 1.0000x reference)
import jax
import jax.numpy as jnp
from jax.experimental import pallas as pl


def kernel(attn_s):
    raise NotImplementedError("write your pallas kernel here")


# speedup vs baseline: 77.8606x; 77.8606x over previous
import functools

import jax
import jax.numpy as jnp
from jax.experimental import pallas as pl
from jax.experimental.pallas import tpu as pltpu

_TOPK1 = 65
_EPS = 1e-07


def _select_body(x_ref, o_ref):
    x = x_ref[...]
    rows = x.shape[0]

    ix = jax.lax.bitcast_convert_type(x, jnp.int32)
    key = jnp.where(ix >= 0, ix, ix ^ jnp.int32(0x7FFFFFFF))
    v = key ^ jnp.int32(-2147483648)

    need = jnp.full((rows, 1), _TOPK1, dtype=jnp.int32)
    p = jnp.zeros((rows, 1), dtype=jnp.int32)
    for _ in range(32):
        msk = v < 0
        cnt = jnp.sum(msk.astype(jnp.int32), axis=1, keepdims=True)
        take = cnt >= need
        p = p * 2 + take.astype(jnp.int32)
        need = need - jnp.where(take, 0, cnt)
        keep = msk == take
        v = jnp.where(keep, v << 1, 0)

    key_t = p ^ jnp.int32(-2147483648)
    it = jnp.where(key_t >= 0, key_t, key_t ^ jnp.int32(0x7FFFFFFF))
    delta = jax.lax.bitcast_convert_type(it, jnp.float32)

    w = jnp.maximum(x - delta, 0.0)
    s = jnp.sum(w, axis=1, keepdims=True) + jnp.float32(_EPS)
    o_ref[...] = w / s


@jax.jit
def kernel(attn_s):
    b, one, n = attn_s.shape
    x = attn_s.reshape(b, n)
    out = pl.pallas_call(
        _select_body,
        out_shape=jax.ShapeDtypeStruct((b, n), jnp.float32),
    )(x)
    return out.reshape(b, one, n)
